# baseline (speedup 1.0000x reference)
import functools

import jax
import jax.numpy as jnp
from jax import lax
from jax.experimental import pallas as pl
from jax.experimental.pallas import tpu as pltpu
from jax.experimental.pallas import tpu_sc as plsc

_B = 4096
_D = 512
_L = 16
_NC = 2
_NS = 16
_NW = _NC * _NS
_ROWS = _B // _NW

_CHUNKS = (8, 24, 32, 32, 32)
_OFFS = (0, 8, 32, 64, 96)
_BOFFS = (0, 8, 32, 64, 0)
_FB = 96
_SEMS = (0, 1, 2, 3, 0)


def _mse_body(feat_hbm, lab_hbm, cent_hbm, out_hbm, buf_v, idx_v, acc_v,
              *sems):
    wid = lax.axis_index("s") * _NC + lax.axis_index("c")
    base = wid * _ROWS
    pltpu.sync_copy(lab_hbm.at[pl.ds(base, _ROWS)], idx_v)

    def start(c):
        s = sems[_SEMS[c]]
        g = pltpu.async_copy(
            cent_hbm.at[idx_v.at[pl.ds(_OFFS[c], _CHUNKS[c])]],
            buf_v.at[pl.ds(_BOFFS[c], _CHUNKS[c])], s)
        f = pltpu.async_copy(
            feat_hbm.at[pl.ds(base + _OFFS[c], _CHUNKS[c])],
            buf_v.at[pl.ds(_FB + _BOFFS[c], _CHUNKS[c])], s)
        return g, f

    inflight = [start(c) for c in range(4)]
    acc = jnp.zeros((_L,), jnp.float32)
    for c in range(len(_CHUNKS)):
        inflight[c][0].wait()
        inflight[c][1].wait()

        def row_body(r, a, _o=_BOFFS[c]):
            for col in range(0, _D, _L):
                d = buf_v[_FB + _o + r, pl.ds(col, _L)] - \
                    buf_v[_o + r, pl.ds(col, _L)]
                a = d * d + a
            return a

        acc = lax.fori_loop(0, _CHUNKS[c], row_body, acc, unroll=1)
        if c == 1:
            inflight.append(start(4))

    acc_v[...] = acc * (1.0 / (_B * _D))
    pltpu.sync_copy(acc_v, out_hbm.at[wid])


@functools.partial(
    pl.kernel,
    out_type=jax.ShapeDtypeStruct((_NW, _L), jnp.float32),
    mesh=plsc.VectorSubcoreMesh(core_axis_name="c", subcore_axis_name="s"),
    scratch_types=[
        pltpu.VMEM((2 * _FB, _D), jnp.float32),
        pltpu.VMEM((_ROWS,), jnp.int32),
        pltpu.VMEM((_L,), jnp.float32),
    ] + [pltpu.SemaphoreType.DMA] * 4,
)
def _mse_kernel(feat_hbm, lab_hbm, cent_hbm, out_hbm, buf_v, idx_v, acc_v,
                *sems):
    _mse_body(feat_hbm, lab_hbm, cent_hbm, out_hbm, buf_v, idx_v, acc_v,
              *sems)


def kernel(features, labels, centers):
    partials = _mse_kernel(features, labels.astype(jnp.int32), centers)
    return jnp.sum(partials)


# speedup vs baseline: 1.0270x; 1.0270x over previous
import functools

import jax
import jax.numpy as jnp
from jax import lax
from jax.experimental import pallas as pl
from jax.experimental.pallas import tpu as pltpu
from jax.experimental.pallas import tpu_sc as plsc

_B = 4096
_D = 512
_L = 16
_NC = 2
_NS = 16
_NW = _NC * _NS
_ROWS = _B // _NW

_CHUNKS = (8, 24, 32, 32, 32)
_OFFS = (0, 8, 32, 64, 96)
_BOFFS = (0, 8, 32, 64, 0)
_FB = 96
_SEMS = ((0, 1), (2, 3), (4, 5), (6, 7), (0, 1))


def _mse_body(feat_hbm, lab_hbm, cent_hbm, out_hbm, buf_v, idx_v, acc_v,
              *sems):
    wid = lax.axis_index("s") * _NC + lax.axis_index("c")
    base = wid * _ROWS
    pltpu.sync_copy(lab_hbm.at[pl.ds(base, _ROWS)], idx_v)

    def start(c):
        sg, sf = _SEMS[c]
        g = pltpu.async_copy(
            cent_hbm.at[idx_v.at[pl.ds(_OFFS[c], _CHUNKS[c])]],
            buf_v.at[pl.ds(_BOFFS[c], _CHUNKS[c])], sems[sg])
        f = pltpu.async_copy(
            feat_hbm.at[pl.ds(base + _OFFS[c], _CHUNKS[c])],
            buf_v.at[pl.ds(_FB + _BOFFS[c], _CHUNKS[c])], sems[sf])
        return g, f

    inflight = [start(c) for c in range(4)]
    acc = jnp.zeros((_L,), jnp.float32)
    for c in range(len(_CHUNKS)):
        inflight[c][0].wait()
        inflight[c][1].wait()

        def row_body(r, a, _o=_BOFFS[c]):
            for col in range(0, _D, _L):
                d = buf_v[_FB + _o + r, pl.ds(col, _L)] - \
                    buf_v[_o + r, pl.ds(col, _L)]
                a = d * d + a
            return a

        acc = lax.fori_loop(0, _CHUNKS[c], row_body, acc, unroll=1)
        if c == 1:
            inflight.append(start(4))

    acc_v[...] = acc * (1.0 / (_B * _D))
    pltpu.sync_copy(acc_v, out_hbm.at[wid])


@functools.partial(
    pl.kernel,
    out_type=jax.ShapeDtypeStruct((_NW, _L), jnp.float32),
    mesh=plsc.VectorSubcoreMesh(core_axis_name="c", subcore_axis_name="s"),
    scratch_types=[
        pltpu.VMEM((2 * _FB, _D), jnp.float32),
        pltpu.VMEM((_ROWS,), jnp.int32),
        pltpu.VMEM((_L,), jnp.float32),
    ] + [pltpu.SemaphoreType.DMA] * 8,
)
def _mse_kernel(feat_hbm, lab_hbm, cent_hbm, out_hbm, buf_v, idx_v, acc_v,
                *sems):
    _mse_body(feat_hbm, lab_hbm, cent_hbm, out_hbm, buf_v, idx_v, acc_v,
              *sems)


def kernel(features, labels, centers):
    partials = _mse_kernel(features, labels.astype(jnp.int32), centers)
    return jnp.sum(partials)
